# baseline (speedup 1.0000x reference)
import functools

import jax
import jax.numpy as jnp
from jax import lax
from jax.experimental import pallas as pl
from jax.experimental.pallas import tpu as pltpu
from jax.experimental.pallas import tpu_sc as plsc

N = 10000
E = 320000
D = 128
HID = 128
G = 64

NC, NS = 2, 16
NW = NC * NS
N_PAD = 10240
PAD_NODE = N_PAD - 1
CH = 128
CPW = 80
SCH = 8
EPW = CPW * CH
E_PAD = NW * EPW
EROWS = E_PAD // CH
RPW = N_PAD // NW
RPS = N_PAD // NS
GCH = 80

_sc_mesh = plsc.VectorSubcoreMesh(
    core_axis_name="c", subcore_axis_name="s", num_cores=NC, num_subcores=NS)


def _mm_body(a_ref, w_ref, o_ref):
    o_ref[...] = jnp.dot(a_ref[...], w_ref[...],
                         preferred_element_type=jnp.float32)


def _matmul(a, w, br):
    m, k = a.shape
    n = w.shape[1]
    return pl.pallas_call(
        _mm_body,
        grid=(m // br,),
        in_specs=[pl.BlockSpec((br, k), lambda i: (i, 0)),
                  pl.BlockSpec((k, n), lambda i: (0, 0))],
        out_specs=pl.BlockSpec((br, n), lambda i: (i, 0)),
        out_shape=jax.ShapeDtypeStruct((m, n), jnp.float32),
    )(a, w)


def _deg_sc_body(dstm_hbm, drnl_hbm, embw_hbm, xw_hbm, degp_hbm, h1pre_hbm,
                 didx, ones_v, gidx, grow, xrow, acc, gsem):
    c = lax.axis_index("c")
    s = lax.axis_index("s")

    def zb(j, carry):
        for t in range(HID // 16):
            ones_v[j, pl.ds(t * 16, 16)] = jnp.zeros((16,), jnp.float32)
        return carry
    lax.fori_loop(0, CH, zb, 0)

    def zc(r, carry):
        pltpu.sync_copy(ones_v, acc.at[pl.ds(s * RPS + r * CH, CH)])
        return carry
    lax.fori_loop(0, RPS // CH, zc, 0)

    def ob(j, carry):
        for t in range(HID // 16):
            ones_v[j, pl.ds(t * 16, 16)] = jnp.ones((16,), jnp.float32)
        return carry
    lax.fori_loop(0, CH, ob, 0)
    plsc.subcore_barrier()

    ebase = c * (E_PAD // 2) + s * EPW

    def eb(i, carry):
        pltpu.sync_copy(dstm_hbm.at[pl.ds(ebase + i * CH, CH)], didx)
        pltpu.sync_copy(ones_v, acc.at[didx], add=True)
        return carry
    lax.fori_loop(0, CPW, eb, 0)

    plsc.subcore_barrier()
    pltpu.sync_copy(acc.at[pl.ds(s * RPS, RPS)],
                    degp_hbm.at[pl.ds(c * N_PAD + s * RPS, RPS)])

    wid = s * NC + c
    nbase = wid * RPW
    for k in range(RPW // GCH):
        pltpu.sync_copy(drnl_hbm.at[pl.ds(nbase + k * GCH, GCH)], gidx)
        pltpu.async_copy(embw_hbm.at[gidx], grow, gsem).wait()
        pltpu.sync_copy(xw_hbm.at[pl.ds(nbase + k * GCH, GCH)], xrow)

        def ab(j, carry):
            for t in range(HID // 16):
                grow[j, pl.ds(t * 16, 16)] = (
                    grow[j, pl.ds(t * 16, 16)] + xrow[j, pl.ds(t * 16, 16)])
            return carry
        lax.fori_loop(0, GCH, ab, 0)
        pltpu.sync_copy(grow, h1pre_hbm.at[pl.ds(nbase + k * GCH, GCH)])


_sc_deg = pl.kernel(
    _deg_sc_body,
    out_type=(jax.ShapeDtypeStruct((2 * N_PAD, HID), jnp.float32),
              jax.ShapeDtypeStruct((N_PAD, HID), jnp.float32)),
    mesh=_sc_mesh,
    scratch_types=[
        pltpu.VMEM((CH,), jnp.int32),
        pltpu.VMEM((CH, HID), jnp.float32),
        pltpu.VMEM((GCH,), jnp.int32),
        pltpu.VMEM((GCH, HID), jnp.float32),
        pltpu.VMEM((GCH, HID), jnp.float32),
        pltpu.VMEM_SHARED((N_PAD, HID), jnp.float32),
        pltpu.SemaphoreType.DMA,
    ],
)


def _deg_body(d0_ref, d1_ref, h_ref, dinv_ref, hs_ref):
    dinv = lax.rsqrt(d0_ref[:, :1] + d1_ref[:, :1] + 1.0)
    dinv_ref[...] = dinv
    hs_ref[...] = h_ref[...] * dinv


def _deg_reduce(degp, h1pre, br=1024):
    nblk = N_PAD // br
    return pl.pallas_call(
        _deg_body,
        grid=(nblk,),
        in_specs=[pl.BlockSpec((br, HID), lambda i: (i, 0)),
                  pl.BlockSpec((br, HID), lambda i: (i + nblk, 0)),
                  pl.BlockSpec((br, HID), lambda i: (i, 0))],
        out_specs=(pl.BlockSpec((br, 1), lambda i: (i, 0)),
                   pl.BlockSpec((br, HID), lambda i: (i, 0))),
        out_shape=(jax.ShapeDtypeStruct((N_PAD, 1), jnp.float32),
                   jax.ShapeDtypeStruct((N_PAD, HID), jnp.float32)),
    )(degp, degp, h1pre)


def _agg_body(hs_hbm, srcm_hbm, dstm_hbm, accp_hbm,
              sidx, didx, rows_a, acc, sem_a):
    c = lax.axis_index("c")
    s = lax.axis_index("s")

    def zb(j, carry):
        for t in range(HID // 16):
            rows_a[j, pl.ds(t * 16, 16)] = jnp.zeros((16,), jnp.float32)
        return carry
    lax.fori_loop(0, CH, zb, 0)

    def zc(r, carry):
        pltpu.sync_copy(rows_a, acc.at[pl.ds(s * RPS + r * CH, CH)])
        return carry
    lax.fori_loop(0, RPS // CH, zc, 0)
    plsc.subcore_barrier()

    ebase = c * (E_PAD // 2) + s * EPW

    def eb(i, carry):
        pltpu.sync_copy(srcm_hbm.at[pl.ds(ebase + i * CH, CH)], sidx)
        pltpu.async_copy(hs_hbm.at[sidx], rows_a, sem_a).wait()
        pltpu.sync_copy(dstm_hbm.at[pl.ds(ebase + i * CH, CH)], didx)
        pltpu.sync_copy(rows_a, acc.at[didx], add=True)
        return carry
    lax.fori_loop(0, CPW, eb, 0)

    plsc.subcore_barrier()
    pltpu.sync_copy(acc.at[pl.ds(s * RPS, RPS)],
                    accp_hbm.at[pl.ds(c * N_PAD + s * RPS, RPS)])


_sc_agg = pl.kernel(
    _agg_body,
    out_type=jax.ShapeDtypeStruct((2 * N_PAD, HID), jnp.float32),
    mesh=_sc_mesh,
    scratch_types=[
        pltpu.VMEM((CH,), jnp.int32),
        pltpu.VMEM((CH,), jnp.int32),
        pltpu.VMEM((CH, HID), jnp.float32),
        pltpu.VMEM_SHARED((N_PAD, HID), jnp.float32),
        pltpu.SemaphoreType.DMA,
    ],
)


def _conv_body(a0_ref, a1_ref, hs_ref, d_ref, b_ref, w_ref, o_ref):
    dinv = d_ref[...]
    z = jnp.maximum(
        dinv * (a0_ref[...] + a1_ref[...] + hs_ref[...]) + b_ref[...], 0.0)
    o_ref[...] = jnp.dot(z, w_ref[...],
                         preferred_element_type=jnp.float32) * dinv


def _conv_mm(accp, hs, dinv_col, brow, w, br=1024):
    nblk = N_PAD // br
    return pl.pallas_call(
        _conv_body,
        grid=(nblk,),
        in_specs=[pl.BlockSpec((br, HID), lambda i: (i, 0)),
                  pl.BlockSpec((br, HID), lambda i: (i + nblk, 0)),
                  pl.BlockSpec((br, HID), lambda i: (i, 0)),
                  pl.BlockSpec((br, 1), lambda i: (i, 0)),
                  pl.BlockSpec((1, HID), lambda i: (0, 0)),
                  pl.BlockSpec((HID, HID), lambda i: (0, 0))],
        out_specs=pl.BlockSpec((br, HID), lambda i: (i, 0)),
        out_shape=jax.ShapeDtypeStruct((N_PAD, HID), jnp.float32),
    )(accp, accp, hs, dinv_col, brow, w)


NBLK2 = N_PAD // 128


def _final_body(acc_ref, hs_ref, dinv_ref, batch_ref, b2d_ref, uv_ref, b2_ref,
                wm1_ref, bm1_ref, wm2_ref, bm2_ref, o_ref, z2_ref, bm_ref):
    dinv = dinv_ref[...]
    z2_ref[...] = jnp.maximum(
        dinv * (acc_ref[:N_PAD, :] + acc_ref[N_PAD:, :] + hs_ref[...])
        + b2_ref[...], 0.0)

    for b in range(NBLK2):
        bm_ref[b:b + 1, :] = jnp.max(
            z2_ref[b * 128:(b + 1) * 128, :], axis=0, keepdims=True)

    gsel = lax.broadcasted_iota(jnp.int32, (G, 1), 0)
    sub8 = lax.broadcasted_iota(jnp.int32, (8, 1), 0)
    blockid = lax.broadcasted_iota(jnp.int32, (NBLK2, 1), 0)

    def _row_at(i):
        base = pl.multiple_of((i // 8) * 8, 8)
        blk = z2_ref[pl.ds(base, 8), :]
        return jnp.max(jnp.where(sub8 == i - base, blk, -jnp.inf), axis=0)

    def _masked_block_max(row0, g):
        blk = z2_ref[pl.ds(row0, 128), :]
        m = batch_ref[pl.ds(row0, 128), :] == g
        return jnp.max(jnp.where(m, blk, -jnp.inf), axis=0)

    def gloop(g, feats):
        batch2d = b2d_ref[...]
        ptr_g = jnp.sum((batch2d < g).astype(jnp.int32))
        ptr_n = jnp.sum((batch2d < g + 1).astype(jnp.int32))
        fullmask = (blockid * 128 >= ptr_g) & ((blockid + 1) * 128 <= ptr_n)
        pool = jnp.max(jnp.where(fullmask, bm_ref[...], -jnp.inf), axis=0)
        lo = pl.multiple_of((ptr_g // 128) * 128, 128)
        hi = pl.multiple_of((jnp.maximum(ptr_n, 1) - 1) // 128 * 128, 128)
        pool = jnp.maximum(pool, _masked_block_max(lo, g))
        pool = jnp.maximum(pool, _masked_block_max(hi, g))
        iu = jnp.minimum(ptr_g + uv_ref[2 * g], N - 1)
        iv = jnp.minimum(ptr_g + uv_ref[2 * g + 1], N - 1)
        hu = _row_at(iu)
        hv = _row_at(iv)
        row = jnp.concatenate(
            [hu, hv, jnp.abs(hu - hv), hu * hv, pool])[None, :]
        return jnp.where(gsel == g, row, feats)

    feats = lax.fori_loop(0, G, gloop,
                          jnp.zeros((G, 5 * HID), jnp.float32))

    hidden = jnp.maximum(
        jnp.dot(feats, wm1_ref[...],
                preferred_element_type=jnp.float32) + bm1_ref[...], 0.0)
    o_ref[...] = jnp.dot(hidden, wm2_ref[...],
                         preferred_element_type=jnp.float32) + bm2_ref[...]


def _final(accp, hs, dinv_col, batch_col, batch2d, uv, b2r, wm1, bm1r, wm2,
           bm2r):
    return pl.pallas_call(
        _final_body,
        in_specs=[
            pl.BlockSpec((2 * N_PAD, HID), lambda: (0, 0)),
            pl.BlockSpec((N_PAD, HID), lambda: (0, 0)),
            pl.BlockSpec((N_PAD, 1), lambda: (0, 0)),
            pl.BlockSpec((N_PAD, 1), lambda: (0, 0)),
            pl.BlockSpec((NBLK2, 128), lambda: (0, 0)),
            pl.BlockSpec(memory_space=pltpu.SMEM),
            pl.BlockSpec((1, HID), lambda: (0, 0)),
            pl.BlockSpec((5 * HID, HID), lambda: (0, 0)),
            pl.BlockSpec((1, HID), lambda: (0, 0)),
            pl.BlockSpec((HID, 2), lambda: (0, 0)),
            pl.BlockSpec((1, 2), lambda: (0, 0)),
        ],
        out_specs=pl.BlockSpec((G, 2), lambda: (0, 0)),
        out_shape=jax.ShapeDtypeStruct((G, 2), jnp.float32),
        scratch_shapes=[pltpu.VMEM((N_PAD, HID), jnp.float32),
                        pltpu.VMEM((NBLK2, 128), jnp.float32)],
    )(accp, hs, dinv_col, batch_col, batch2d, uv, b2r, wm1, bm1r, wm2, bm2r)


def kernel(x, edge_index, drnl, batch, target_local, emb, W1, b1, W2, b2,
           Wm1, bm1, Wm2, bm2):
    src = edge_index[0].astype(jnp.int32)
    dst = edge_index[1].astype(jnp.int32)
    src_p = jnp.full((E_PAD,), PAD_NODE, jnp.int32).at[:E].set(src)
    dst_p = jnp.full((E_PAD,), PAD_NODE, jnp.int32).at[:E].set(dst)
    drnl_p = jnp.zeros((N_PAD,), jnp.int32).at[:N].set(drnl.astype(jnp.int32))
    batch_col = jnp.full((N_PAD, 1), G, jnp.int32).at[:N, 0].set(
        batch.astype(jnp.int32))
    batch2d = batch_col.reshape(NBLK2, 128)
    x_p = jnp.zeros((N_PAD, D), jnp.float32).at[:N].set(x)
    uv = target_local.astype(jnp.int32)

    embw = _matmul(emb, W1[D:], emb.shape[0])
    xw = _matmul(x_p, W1[:D], 1024)
    degp, h1pre = _sc_deg(dst_p, drnl_p, embw, xw)
    dinv_col, hs1 = _deg_reduce(degp, h1pre)
    accp1 = _sc_agg(hs1, src_p, dst_p)
    hs2 = _conv_mm(accp1, hs1, dinv_col, b1.reshape(1, HID), W2)
    accp2 = _sc_agg(hs2, src_p, dst_p)
    return _final(accp2, hs2, dinv_col, batch_col, batch2d, uv,
                  b2.reshape(1, HID), Wm1, bm1.reshape(1, HID), Wm2,
                  bm2.reshape(1, 2))


# speedup vs baseline: 2.0295x; 2.0295x over previous
import functools

import jax
import jax.numpy as jnp
from jax import lax
from jax.experimental import pallas as pl
from jax.experimental.pallas import tpu as pltpu
from jax.experimental.pallas import tpu_sc as plsc

N = 10000
E = 320000
D = 128
HID = 128
G = 64

NC, NS = 2, 16
NW = NC * NS
N_PAD = 10240
PAD_NODE = N_PAD - 1
CH = 128
CPW = 80
SCH = 8
EPW = CPW * CH
E_PAD = NW * EPW
EROWS = E_PAD // CH
RPW = N_PAD // NW
RPS = N_PAD // NS
GCH = 80

_sc_mesh = plsc.VectorSubcoreMesh(
    core_axis_name="c", subcore_axis_name="s", num_cores=NC, num_subcores=NS)


def _mm_body(a_ref, w_ref, o_ref):
    o_ref[...] = jnp.dot(a_ref[...], w_ref[...],
                         preferred_element_type=jnp.float32)


def _matmul(a, w, br):
    m, k = a.shape
    n = w.shape[1]
    return pl.pallas_call(
        _mm_body,
        grid=(m // br,),
        in_specs=[pl.BlockSpec((br, k), lambda i: (i, 0)),
                  pl.BlockSpec((k, n), lambda i: (0, 0))],
        out_specs=pl.BlockSpec((br, n), lambda i: (i, 0)),
        out_shape=jax.ShapeDtypeStruct((m, n), jnp.float32),
    )(a, w)


def _deg_sc_body(dstm_hbm, drnl_hbm, embw_hbm, xw_hbm, degp_hbm, h1pre_hbm,
                 didx, ones_v, gidx, grow, xrow, acc, gsem):
    c = lax.axis_index("c")
    s = lax.axis_index("s")

    def zb(j, carry):
        for t in range(HID // 16):
            ones_v[j, pl.ds(t * 16, 16)] = jnp.zeros((16,), jnp.float32)
        return carry
    lax.fori_loop(0, CH, zb, 0)

    def zc(r, carry):
        pltpu.sync_copy(ones_v, acc.at[pl.ds(s * RPS + r * CH, CH)])
        return carry
    lax.fori_loop(0, RPS // CH, zc, 0)

    def ob(j, carry):
        for t in range(HID // 16):
            ones_v[j, pl.ds(t * 16, 16)] = jnp.ones((16,), jnp.float32)
        return carry
    lax.fori_loop(0, CH, ob, 0)
    plsc.subcore_barrier()

    ebase = c * (E_PAD // 2) + s * EPW

    def eb(i, carry):
        pltpu.sync_copy(dstm_hbm.at[pl.ds(ebase + i * CH, CH)], didx)
        pltpu.sync_copy(ones_v, acc.at[didx], add=True)
        return carry
    lax.fori_loop(0, CPW, eb, 0)

    plsc.subcore_barrier()
    pltpu.sync_copy(acc.at[pl.ds(s * RPS, RPS)],
                    degp_hbm.at[pl.ds(c * N_PAD + s * RPS, RPS)])

    wid = s * NC + c
    nbase = wid * RPW
    for k in range(RPW // GCH):
        pltpu.sync_copy(drnl_hbm.at[pl.ds(nbase + k * GCH, GCH)], gidx)
        pltpu.async_copy(embw_hbm.at[gidx], grow, gsem).wait()
        pltpu.sync_copy(xw_hbm.at[pl.ds(nbase + k * GCH, GCH)], xrow)

        def ab(j, carry):
            for t in range(HID // 16):
                grow[j, pl.ds(t * 16, 16)] = (
                    grow[j, pl.ds(t * 16, 16)] + xrow[j, pl.ds(t * 16, 16)])
            return carry
        lax.fori_loop(0, GCH, ab, 0)
        pltpu.sync_copy(grow, h1pre_hbm.at[pl.ds(nbase + k * GCH, GCH)])


_sc_deg = pl.kernel(
    _deg_sc_body,
    out_type=(jax.ShapeDtypeStruct((2 * N_PAD, HID), jnp.float32),
              jax.ShapeDtypeStruct((N_PAD, HID), jnp.float32)),
    mesh=_sc_mesh,
    scratch_types=[
        pltpu.VMEM((CH,), jnp.int32),
        pltpu.VMEM((CH, HID), jnp.float32),
        pltpu.VMEM((GCH,), jnp.int32),
        pltpu.VMEM((GCH, HID), jnp.float32),
        pltpu.VMEM((GCH, HID), jnp.float32),
        pltpu.VMEM_SHARED((N_PAD, HID), jnp.float32),
        pltpu.SemaphoreType.DMA,
    ],
)


def _deg_body(d0_ref, d1_ref, h_ref, dinv_ref, hs_ref):
    dinv = lax.rsqrt(d0_ref[:, :1] + d1_ref[:, :1] + 1.0)
    dinv_ref[...] = dinv
    hs_ref[...] = h_ref[...] * dinv


def _deg_reduce(degp, h1pre, br=1024):
    nblk = N_PAD // br
    return pl.pallas_call(
        _deg_body,
        grid=(nblk,),
        in_specs=[pl.BlockSpec((br, HID), lambda i: (i, 0)),
                  pl.BlockSpec((br, HID), lambda i: (i + nblk, 0)),
                  pl.BlockSpec((br, HID), lambda i: (i, 0))],
        out_specs=(pl.BlockSpec((br, 1), lambda i: (i, 0)),
                   pl.BlockSpec((br, HID), lambda i: (i, 0))),
        out_shape=(jax.ShapeDtypeStruct((N_PAD, 1), jnp.float32),
                   jax.ShapeDtypeStruct((N_PAD, HID), jnp.float32)),
    )(degp, degp, h1pre)


def _agg_body(hs_hbm, srcm_hbm, dstm_hbm, accp_hbm,
              sidx, didx, rows_a, acc, sem_a):
    c = lax.axis_index("c")
    s = lax.axis_index("s")

    def zb(j, carry):
        for t in range(HID // 16):
            rows_a[j, pl.ds(t * 16, 16)] = jnp.zeros((16,), jnp.float32)
        return carry
    lax.fori_loop(0, CH, zb, 0)

    def zc(r, carry):
        pltpu.sync_copy(rows_a, acc.at[pl.ds(s * RPS + r * CH, CH)])
        return carry
    lax.fori_loop(0, RPS // CH, zc, 0)
    plsc.subcore_barrier()

    ebase = c * (E_PAD // 2) + s * EPW

    def eb(i, carry):
        pltpu.sync_copy(srcm_hbm.at[pl.ds(ebase + i * CH, CH)], sidx)
        pltpu.async_copy(hs_hbm.at[sidx], rows_a, sem_a).wait()
        pltpu.sync_copy(dstm_hbm.at[pl.ds(ebase + i * CH, CH)], didx)
        pltpu.sync_copy(rows_a, acc.at[didx], add=True)
        return carry
    lax.fori_loop(0, CPW, eb, 0)

    plsc.subcore_barrier()
    pltpu.sync_copy(acc.at[pl.ds(s * RPS, RPS)],
                    accp_hbm.at[pl.ds(c * N_PAD + s * RPS, RPS)])


_sc_agg = pl.kernel(
    _agg_body,
    out_type=jax.ShapeDtypeStruct((2 * N_PAD, HID), jnp.float32),
    mesh=_sc_mesh,
    scratch_types=[
        pltpu.VMEM((CH,), jnp.int32),
        pltpu.VMEM((CH,), jnp.int32),
        pltpu.VMEM((CH, HID), jnp.float32),
        pltpu.VMEM_SHARED((N_PAD, HID), jnp.float32),
        pltpu.SemaphoreType.DMA,
    ],
)


def _conv_body(a0_ref, a1_ref, hs_ref, d_ref, b_ref, w_ref, o_ref):
    dinv = d_ref[...]
    z = jnp.maximum(
        dinv * (a0_ref[...] + a1_ref[...] + hs_ref[...]) + b_ref[...], 0.0)
    o_ref[...] = jnp.dot(z, w_ref[...],
                         preferred_element_type=jnp.float32) * dinv


def _conv_mm(accp, hs, dinv_col, brow, w, br=1024):
    nblk = N_PAD // br
    return pl.pallas_call(
        _conv_body,
        grid=(nblk,),
        in_specs=[pl.BlockSpec((br, HID), lambda i: (i, 0)),
                  pl.BlockSpec((br, HID), lambda i: (i + nblk, 0)),
                  pl.BlockSpec((br, HID), lambda i: (i, 0)),
                  pl.BlockSpec((br, 1), lambda i: (i, 0)),
                  pl.BlockSpec((1, HID), lambda i: (0, 0)),
                  pl.BlockSpec((HID, HID), lambda i: (0, 0))],
        out_specs=pl.BlockSpec((br, HID), lambda i: (i, 0)),
        out_shape=jax.ShapeDtypeStruct((N_PAD, HID), jnp.float32),
    )(accp, accp, hs, dinv_col, brow, w)


NBLK2 = N_PAD // 128


def _final_body(acc_ref, hs_ref, dinv_ref, batch_ref, b2d_ref, uv_ref, b2_ref,
                wm1_ref, bm1_ref, wm2_ref, bm2_ref, o_ref, z2_ref, bm_ref):
    dinv = dinv_ref[...]
    z2_ref[...] = jnp.maximum(
        dinv * (acc_ref[:N_PAD, :] + acc_ref[N_PAD:, :] + hs_ref[...])
        + b2_ref[...], 0.0)

    for b in range(NBLK2):
        bm_ref[b:b + 1, :] = jnp.max(
            z2_ref[b * 128:(b + 1) * 128, :], axis=0, keepdims=True)

    gsel = lax.broadcasted_iota(jnp.int32, (G, 1), 0)
    sub8 = lax.broadcasted_iota(jnp.int32, (8, 1), 0)
    blockid = lax.broadcasted_iota(jnp.int32, (NBLK2, 1), 0)

    def _row_at(i):
        base = pl.multiple_of((i // 8) * 8, 8)
        blk = z2_ref[pl.ds(base, 8), :]
        return jnp.max(jnp.where(sub8 == i - base, blk, -jnp.inf), axis=0)

    def _masked_block_max(row0, g):
        blk = z2_ref[pl.ds(row0, 128), :]
        m = batch_ref[pl.ds(row0, 128), :] == g
        return jnp.max(jnp.where(m, blk, -jnp.inf), axis=0)

    def gloop(g, feats):
        batch2d = b2d_ref[...]
        ptr_g = jnp.sum((batch2d < g).astype(jnp.int32))
        ptr_n = jnp.sum((batch2d < g + 1).astype(jnp.int32))
        fullmask = (blockid * 128 >= ptr_g) & ((blockid + 1) * 128 <= ptr_n)
        pool = jnp.max(jnp.where(fullmask, bm_ref[...], -jnp.inf), axis=0)
        lo = pl.multiple_of((ptr_g // 128) * 128, 128)
        hi = pl.multiple_of((jnp.maximum(ptr_n, 1) - 1) // 128 * 128, 128)
        pool = jnp.maximum(pool, _masked_block_max(lo, g))
        pool = jnp.maximum(pool, _masked_block_max(hi, g))
        iu = jnp.minimum(ptr_g + uv_ref[2 * g], N - 1)
        iv = jnp.minimum(ptr_g + uv_ref[2 * g + 1], N - 1)
        hu = _row_at(iu)
        hv = _row_at(iv)
        row = jnp.concatenate(
            [hu, hv, jnp.abs(hu - hv), hu * hv, pool])[None, :]
        return jnp.where(gsel == g, row, feats)

    feats = lax.fori_loop(0, G, gloop,
                          jnp.zeros((G, 5 * HID), jnp.float32))

    hidden = jnp.maximum(
        jnp.dot(feats, wm1_ref[...],
                preferred_element_type=jnp.float32) + bm1_ref[...], 0.0)
    o_ref[...] = jnp.dot(hidden, wm2_ref[...],
                         preferred_element_type=jnp.float32) + bm2_ref[...]


def _final(accp, hs, dinv_col, batch_col, batch2d, uv, b2r, wm1, bm1r, wm2,
           bm2r):
    return pl.pallas_call(
        _final_body,
        in_specs=[
            pl.BlockSpec((2 * N_PAD, HID), lambda: (0, 0)),
            pl.BlockSpec((N_PAD, HID), lambda: (0, 0)),
            pl.BlockSpec((N_PAD, 1), lambda: (0, 0)),
            pl.BlockSpec((N_PAD, 1), lambda: (0, 0)),
            pl.BlockSpec((NBLK2, 128), lambda: (0, 0)),
            pl.BlockSpec(memory_space=pltpu.SMEM),
            pl.BlockSpec((1, HID), lambda: (0, 0)),
            pl.BlockSpec((5 * HID, HID), lambda: (0, 0)),
            pl.BlockSpec((1, HID), lambda: (0, 0)),
            pl.BlockSpec((HID, 2), lambda: (0, 0)),
            pl.BlockSpec((1, 2), lambda: (0, 0)),
        ],
        out_specs=pl.BlockSpec((G, 2), lambda: (0, 0)),
        out_shape=jax.ShapeDtypeStruct((G, 2), jnp.float32),
        scratch_shapes=[pltpu.VMEM((N_PAD, HID), jnp.float32),
                        pltpu.VMEM((NBLK2, 128), jnp.float32)],
    )(accp, hs, dinv_col, batch_col, batch2d, uv, b2r, wm1, bm1r, wm2, bm2r)


def kernel(x, edge_index, drnl, batch, target_local, emb, W1, b1, W2, b2,
           Wm1, bm1, Wm2, bm2):
    src = edge_index[0].astype(jnp.int32)
    dst = edge_index[1].astype(jnp.int32)
    pad_fill = N + (jnp.arange(E_PAD - E, dtype=jnp.int32) % (N_PAD - N))
    src_p = jnp.concatenate([src, pad_fill])
    dst_p = jnp.concatenate([dst, pad_fill])
    drnl_p = jnp.zeros((N_PAD,), jnp.int32).at[:N].set(drnl.astype(jnp.int32))
    batch_col = jnp.full((N_PAD, 1), G, jnp.int32).at[:N, 0].set(
        batch.astype(jnp.int32))
    batch2d = batch_col.reshape(NBLK2, 128)
    x_p = jnp.zeros((N_PAD, D), jnp.float32).at[:N].set(x)
    uv = target_local.astype(jnp.int32)

    embw = _matmul(emb, W1[D:], emb.shape[0])
    xw = _matmul(x_p, W1[:D], 1024)
    degp, h1pre = _sc_deg(dst_p, drnl_p, embw, xw)
    dinv_col, hs1 = _deg_reduce(degp, h1pre)
    accp1 = _sc_agg(hs1, src_p, dst_p)
    hs2 = _conv_mm(accp1, hs1, dinv_col, b1.reshape(1, HID), W2)
    accp2 = _sc_agg(hs2, src_p, dst_p)
    return _final(accp2, hs2, dinv_col, batch_col, batch2d, uv,
                  b2.reshape(1, HID), Wm1, bm1.reshape(1, HID), Wm2,
                  bm2.reshape(1, 2))


# speedup vs baseline: 2.5682x; 1.2654x over previous
import functools

import jax
import jax.numpy as jnp
from jax import lax
from jax.experimental import pallas as pl
from jax.experimental.pallas import tpu as pltpu
from jax.experimental.pallas import tpu_sc as plsc

N = 10000
E = 320000
D = 128
HID = 128
G = 64

NC, NS = 2, 16
NW = NC * NS
N_PAD = 10240
PAD_NODE = N_PAD - 1
CH = 128
CPW = 80
SCH = 8
EPW = CPW * CH
E_PAD = NW * EPW
EROWS = E_PAD // CH
RPW = N_PAD // NW
RPS = N_PAD // NS
GCH = 80

_sc_mesh = plsc.VectorSubcoreMesh(
    core_axis_name="c", subcore_axis_name="s", num_cores=NC, num_subcores=NS)


def _mm_body(a_ref, w_ref, o_ref):
    o_ref[...] = jnp.dot(a_ref[...], w_ref[...],
                         preferred_element_type=jnp.float32)


def _matmul(a, w, br):
    m, k = a.shape
    n = w.shape[1]
    return pl.pallas_call(
        _mm_body,
        grid=(m // br,),
        in_specs=[pl.BlockSpec((br, k), lambda i: (i, 0)),
                  pl.BlockSpec((k, n), lambda i: (0, 0))],
        out_specs=pl.BlockSpec((br, n), lambda i: (i, 0)),
        out_shape=jax.ShapeDtypeStruct((m, n), jnp.float32),
    )(a, w)


def _deg_sc_body(dstm_hbm, drnl_hbm, embw_hbm, xw_hbm, degp_hbm, h1pre_hbm,
                 didx, ones_v, gidx, grow, xrow, acc, gsem):
    c = lax.axis_index("c")
    s = lax.axis_index("s")

    def zb(j, carry):
        for t in range(HID // 16):
            ones_v[j, pl.ds(t * 16, 16)] = jnp.zeros((16,), jnp.float32)
        return carry
    lax.fori_loop(0, CH, zb, 0)

    def zc(r, carry):
        pltpu.sync_copy(ones_v, acc.at[pl.ds(s * RPS + r * CH, CH)])
        return carry
    lax.fori_loop(0, RPS // CH, zc, 0)

    def ob(j, carry):
        for t in range(HID // 16):
            ones_v[j, pl.ds(t * 16, 16)] = jnp.ones((16,), jnp.float32)
        return carry
    lax.fori_loop(0, CH, ob, 0)
    plsc.subcore_barrier()

    ebase = c * (E_PAD // 2) + s * EPW

    def eb(i, carry):
        pltpu.sync_copy(dstm_hbm.at[pl.ds(ebase + i * CH, CH)], didx)
        pltpu.sync_copy(ones_v, acc.at[didx], add=True)
        return carry
    lax.fori_loop(0, CPW, eb, 0)

    plsc.subcore_barrier()
    pltpu.sync_copy(acc.at[pl.ds(s * RPS, RPS)],
                    degp_hbm.at[pl.ds(c * N_PAD + s * RPS, RPS)])

    wid = s * NC + c
    nbase = wid * RPW
    for k in range(RPW // GCH):
        pltpu.sync_copy(drnl_hbm.at[pl.ds(nbase + k * GCH, GCH)], gidx)
        pltpu.async_copy(embw_hbm.at[gidx], grow, gsem).wait()
        pltpu.sync_copy(xw_hbm.at[pl.ds(nbase + k * GCH, GCH)], xrow)

        def ab(j, carry):
            for t in range(HID // 16):
                grow[j, pl.ds(t * 16, 16)] = (
                    grow[j, pl.ds(t * 16, 16)] + xrow[j, pl.ds(t * 16, 16)])
            return carry
        lax.fori_loop(0, GCH, ab, 0)
        pltpu.sync_copy(grow, h1pre_hbm.at[pl.ds(nbase + k * GCH, GCH)])


_sc_deg = pl.kernel(
    _deg_sc_body,
    out_type=(jax.ShapeDtypeStruct((2 * N_PAD, HID), jnp.float32),
              jax.ShapeDtypeStruct((N_PAD, HID), jnp.float32)),
    mesh=_sc_mesh,
    scratch_types=[
        pltpu.VMEM((CH,), jnp.int32),
        pltpu.VMEM((CH, HID), jnp.float32),
        pltpu.VMEM((GCH,), jnp.int32),
        pltpu.VMEM((GCH, HID), jnp.float32),
        pltpu.VMEM((GCH, HID), jnp.float32),
        pltpu.VMEM_SHARED((N_PAD, HID), jnp.float32),
        pltpu.SemaphoreType.DMA,
    ],
)


def _deg_body(d0_ref, d1_ref, h_ref, dinv_ref, hs_ref):
    dinv = lax.rsqrt(d0_ref[:, :1] + d1_ref[:, :1] + 1.0)
    dinv_ref[...] = dinv
    hs_ref[...] = h_ref[...] * dinv


def _deg_reduce(degp, h1pre, br=1024):
    nblk = N_PAD // br
    return pl.pallas_call(
        _deg_body,
        grid=(nblk,),
        in_specs=[pl.BlockSpec((br, HID), lambda i: (i, 0)),
                  pl.BlockSpec((br, HID), lambda i: (i + nblk, 0)),
                  pl.BlockSpec((br, HID), lambda i: (i, 0))],
        out_specs=(pl.BlockSpec((br, 1), lambda i: (i, 0)),
                   pl.BlockSpec((br, HID), lambda i: (i, 0))),
        out_shape=(jax.ShapeDtypeStruct((N_PAD, 1), jnp.float32),
                   jax.ShapeDtypeStruct((N_PAD, HID), jnp.float32)),
    )(degp, degp, h1pre)


def _agg_body(hs_hbm, srcm_hbm, dstm_hbm, accp_hbm,
              sidx, didx, sidx_b, didx_b, rows_a, rows_b, acc, sem_a, sem_b):
    c = lax.axis_index("c")
    s = lax.axis_index("s")

    def zb(j, carry):
        for t in range(HID // 16):
            rows_a[j, pl.ds(t * 16, 16)] = jnp.zeros((16,), jnp.float32)
        return carry
    lax.fori_loop(0, CH, zb, 0)

    def zc(r, carry):
        pltpu.sync_copy(rows_a, acc.at[pl.ds(s * RPS + r * CH, CH)])
        return carry
    lax.fori_loop(0, RPS // CH, zc, 0)
    plsc.subcore_barrier()

    ebase = c * (E_PAD // 2) + s * EPW

    def eb(i, carry):
        base = ebase + 2 * i * CH
        pltpu.sync_copy(srcm_hbm.at[pl.ds(base, CH)], sidx)
        cp_a = pltpu.async_copy(hs_hbm.at[sidx], rows_a, sem_a)
        pltpu.sync_copy(srcm_hbm.at[pl.ds(base + CH, CH)], sidx_b)
        cp_b = pltpu.async_copy(hs_hbm.at[sidx_b], rows_b, sem_b)
        pltpu.sync_copy(dstm_hbm.at[pl.ds(base, CH)], didx)
        cp_a.wait()
        pltpu.sync_copy(rows_a, acc.at[didx], add=True)
        pltpu.sync_copy(dstm_hbm.at[pl.ds(base + CH, CH)], didx_b)
        cp_b.wait()
        pltpu.sync_copy(rows_b, acc.at[didx_b], add=True)
        return carry
    lax.fori_loop(0, CPW // 2, eb, 0)

    plsc.subcore_barrier()
    pltpu.sync_copy(acc.at[pl.ds(s * RPS, RPS)],
                    accp_hbm.at[pl.ds(c * N_PAD + s * RPS, RPS)])


_sc_agg = pl.kernel(
    _agg_body,
    out_type=jax.ShapeDtypeStruct((2 * N_PAD, HID), jnp.float32),
    mesh=_sc_mesh,
    scratch_types=[
        pltpu.VMEM((CH,), jnp.int32),
        pltpu.VMEM((CH,), jnp.int32),
        pltpu.VMEM((CH,), jnp.int32),
        pltpu.VMEM((CH,), jnp.int32),
        pltpu.VMEM((CH, HID), jnp.float32),
        pltpu.VMEM((CH, HID), jnp.float32),
        pltpu.VMEM_SHARED((N_PAD, HID), jnp.float32),
        pltpu.SemaphoreType.DMA,
        pltpu.SemaphoreType.DMA,
    ],
)


def _conv_body(a0_ref, a1_ref, hs_ref, d_ref, b_ref, w_ref, o_ref):
    dinv = d_ref[...]
    z = jnp.maximum(
        dinv * (a0_ref[...] + a1_ref[...] + hs_ref[...]) + b_ref[...], 0.0)
    o_ref[...] = jnp.dot(z, w_ref[...],
                         preferred_element_type=jnp.float32) * dinv


def _conv_mm(accp, hs, dinv_col, brow, w, br=1024):
    nblk = N_PAD // br
    return pl.pallas_call(
        _conv_body,
        grid=(nblk,),
        in_specs=[pl.BlockSpec((br, HID), lambda i: (i, 0)),
                  pl.BlockSpec((br, HID), lambda i: (i + nblk, 0)),
                  pl.BlockSpec((br, HID), lambda i: (i, 0)),
                  pl.BlockSpec((br, 1), lambda i: (i, 0)),
                  pl.BlockSpec((1, HID), lambda i: (0, 0)),
                  pl.BlockSpec((HID, HID), lambda i: (0, 0))],
        out_specs=pl.BlockSpec((br, HID), lambda i: (i, 0)),
        out_shape=jax.ShapeDtypeStruct((N_PAD, HID), jnp.float32),
    )(accp, accp, hs, dinv_col, brow, w)


NBLK2 = N_PAD // 128


def _final_body(acc_ref, hs_ref, dinv_ref, batch_ref, b2d_ref, uv_ref, b2_ref,
                wm1_ref, bm1_ref, wm2_ref, bm2_ref, o_ref, z2_ref, bm_ref):
    dinv = dinv_ref[...]
    z2_ref[...] = jnp.maximum(
        dinv * (acc_ref[:N_PAD, :] + acc_ref[N_PAD:, :] + hs_ref[...])
        + b2_ref[...], 0.0)

    for b in range(NBLK2):
        bm_ref[b:b + 1, :] = jnp.max(
            z2_ref[b * 128:(b + 1) * 128, :], axis=0, keepdims=True)

    gsel = lax.broadcasted_iota(jnp.int32, (G, 1), 0)
    sub8 = lax.broadcasted_iota(jnp.int32, (8, 1), 0)
    blockid = lax.broadcasted_iota(jnp.int32, (NBLK2, 1), 0)

    def _row_at(i):
        base = pl.multiple_of((i // 8) * 8, 8)
        blk = z2_ref[pl.ds(base, 8), :]
        return jnp.max(jnp.where(sub8 == i - base, blk, -jnp.inf), axis=0)

    def _masked_block_max(row0, g):
        blk = z2_ref[pl.ds(row0, 128), :]
        m = batch_ref[pl.ds(row0, 128), :] == g
        return jnp.max(jnp.where(m, blk, -jnp.inf), axis=0)

    def gloop(g, feats):
        batch2d = b2d_ref[...]
        ptr_g = jnp.sum((batch2d < g).astype(jnp.int32))
        ptr_n = jnp.sum((batch2d < g + 1).astype(jnp.int32))
        fullmask = (blockid * 128 >= ptr_g) & ((blockid + 1) * 128 <= ptr_n)
        pool = jnp.max(jnp.where(fullmask, bm_ref[...], -jnp.inf), axis=0)
        lo = pl.multiple_of((ptr_g // 128) * 128, 128)
        hi = pl.multiple_of((jnp.maximum(ptr_n, 1) - 1) // 128 * 128, 128)
        pool = jnp.maximum(pool, _masked_block_max(lo, g))
        pool = jnp.maximum(pool, _masked_block_max(hi, g))
        iu = jnp.minimum(ptr_g + uv_ref[2 * g], N - 1)
        iv = jnp.minimum(ptr_g + uv_ref[2 * g + 1], N - 1)
        hu = _row_at(iu)
        hv = _row_at(iv)
        row = jnp.concatenate(
            [hu, hv, jnp.abs(hu - hv), hu * hv, pool])[None, :]
        return jnp.where(gsel == g, row, feats)

    feats = lax.fori_loop(0, G, gloop,
                          jnp.zeros((G, 5 * HID), jnp.float32))

    hidden = jnp.maximum(
        jnp.dot(feats, wm1_ref[...],
                preferred_element_type=jnp.float32) + bm1_ref[...], 0.0)
    o_ref[...] = jnp.dot(hidden, wm2_ref[...],
                         preferred_element_type=jnp.float32) + bm2_ref[...]


def _final(accp, hs, dinv_col, batch_col, batch2d, uv, b2r, wm1, bm1r, wm2,
           bm2r):
    return pl.pallas_call(
        _final_body,
        in_specs=[
            pl.BlockSpec((2 * N_PAD, HID), lambda: (0, 0)),
            pl.BlockSpec((N_PAD, HID), lambda: (0, 0)),
            pl.BlockSpec((N_PAD, 1), lambda: (0, 0)),
            pl.BlockSpec((N_PAD, 1), lambda: (0, 0)),
            pl.BlockSpec((NBLK2, 128), lambda: (0, 0)),
            pl.BlockSpec(memory_space=pltpu.SMEM),
            pl.BlockSpec((1, HID), lambda: (0, 0)),
            pl.BlockSpec((5 * HID, HID), lambda: (0, 0)),
            pl.BlockSpec((1, HID), lambda: (0, 0)),
            pl.BlockSpec((HID, 2), lambda: (0, 0)),
            pl.BlockSpec((1, 2), lambda: (0, 0)),
        ],
        out_specs=pl.BlockSpec((G, 2), lambda: (0, 0)),
        out_shape=jax.ShapeDtypeStruct((G, 2), jnp.float32),
        scratch_shapes=[pltpu.VMEM((N_PAD, HID), jnp.float32),
                        pltpu.VMEM((NBLK2, 128), jnp.float32)],
    )(accp, hs, dinv_col, batch_col, batch2d, uv, b2r, wm1, bm1r, wm2, bm2r)


def kernel(x, edge_index, drnl, batch, target_local, emb, W1, b1, W2, b2,
           Wm1, bm1, Wm2, bm2):
    src = edge_index[0].astype(jnp.int32)
    dst = edge_index[1].astype(jnp.int32)
    pad_fill = N + (jnp.arange(E_PAD - E, dtype=jnp.int32) % (N_PAD - N))
    src_p = jnp.concatenate([src, pad_fill])
    dst_p = jnp.concatenate([dst, pad_fill])
    drnl_p = jnp.zeros((N_PAD,), jnp.int32).at[:N].set(drnl.astype(jnp.int32))
    batch_col = jnp.full((N_PAD, 1), G, jnp.int32).at[:N, 0].set(
        batch.astype(jnp.int32))
    batch2d = batch_col.reshape(NBLK2, 128)
    x_p = jnp.zeros((N_PAD, D), jnp.float32).at[:N].set(x)
    uv = target_local.astype(jnp.int32)

    embw = _matmul(emb, W1[D:], emb.shape[0])
    xw = _matmul(x_p, W1[:D], 1024)
    degp, h1pre = _sc_deg(dst_p, drnl_p, embw, xw)
    dinv_col, hs1 = _deg_reduce(degp, h1pre)
    accp1 = _sc_agg(hs1, src_p, dst_p)
    hs2 = _conv_mm(accp1, hs1, dinv_col, b1.reshape(1, HID), W2)
    accp2 = _sc_agg(hs2, src_p, dst_p)
    return _final(accp2, hs2, dinv_col, batch_col, batch2d, uv,
                  b2.reshape(1, HID), Wm1, bm1.reshape(1, HID), Wm2,
                  bm2.reshape(1, 2))


# speedup vs baseline: 2.7753x; 1.0807x over previous
import functools

import jax
import jax.numpy as jnp
from jax import lax
from jax.experimental import pallas as pl
from jax.experimental.pallas import tpu as pltpu
from jax.experimental.pallas import tpu_sc as plsc

N = 10000
E = 320000
D = 128
HID = 128
G = 64

NC, NS = 2, 16
NW = NC * NS
N_PAD = 10240
PAD_NODE = N_PAD - 1
CH = 128
CPW = 80
SCH = 8
EPW = CPW * CH
E_PAD = NW * EPW
EROWS = E_PAD // CH
RPW = N_PAD // NW
RPS = N_PAD // NS
GCH = 80

_sc_mesh = plsc.VectorSubcoreMesh(
    core_axis_name="c", subcore_axis_name="s", num_cores=NC, num_subcores=NS)


def _mm_body(a_ref, w_ref, o_ref):
    o_ref[...] = jnp.dot(a_ref[...], w_ref[...],
                         preferred_element_type=jnp.float32)


def _matmul(a, w, br):
    m, k = a.shape
    n = w.shape[1]
    return pl.pallas_call(
        _mm_body,
        grid=(m // br,),
        in_specs=[pl.BlockSpec((br, k), lambda i: (i, 0)),
                  pl.BlockSpec((k, n), lambda i: (0, 0))],
        out_specs=pl.BlockSpec((br, n), lambda i: (i, 0)),
        out_shape=jax.ShapeDtypeStruct((m, n), jnp.float32),
    )(a, w)


def _deg_sc_body(dstm_hbm, drnl_hbm, embw_hbm, xw_hbm, degp_hbm, h1pre_hbm,
                 didx, didx_b, ones_v, gidx, grow, xrow, acc, gsem):
    c = lax.axis_index("c")
    s = lax.axis_index("s")

    def zb(j, carry):
        for t in range(HID // 16):
            ones_v[j, pl.ds(t * 16, 16)] = jnp.zeros((16,), jnp.float32)
        return carry
    lax.fori_loop(0, CH, zb, 0)

    def zc(r, carry):
        pltpu.sync_copy(ones_v, acc.at[pl.ds(s * RPS + r * CH, CH)])
        return carry
    lax.fori_loop(0, RPS // CH, zc, 0)

    def ob(j, carry):
        for t in range(HID // 16):
            ones_v[j, pl.ds(t * 16, 16)] = jnp.ones((16,), jnp.float32)
        return carry
    lax.fori_loop(0, CH, ob, 0)
    plsc.subcore_barrier()

    ebase = c * (E_PAD // 2) + s * EPW

    def eb(i, carry):
        base = ebase + 2 * i * CH
        pltpu.sync_copy(dstm_hbm.at[pl.ds(base, CH)], didx)
        cp = pltpu.async_copy(dstm_hbm.at[pl.ds(base + CH, CH)], didx_b, gsem)
        pltpu.sync_copy(ones_v, acc.at[didx], add=True)
        cp.wait()
        pltpu.sync_copy(ones_v, acc.at[didx_b], add=True)
        return carry
    lax.fori_loop(0, CPW // 2, eb, 0)

    plsc.subcore_barrier()
    pltpu.sync_copy(acc.at[pl.ds(s * RPS, RPS)],
                    degp_hbm.at[pl.ds(c * N_PAD + s * RPS, RPS)])

    wid = s * NC + c
    nbase = wid * RPW
    for k in range(RPW // GCH):
        pltpu.sync_copy(drnl_hbm.at[pl.ds(nbase + k * GCH, GCH)], gidx)
        pltpu.async_copy(embw_hbm.at[gidx], grow, gsem).wait()
        pltpu.sync_copy(xw_hbm.at[pl.ds(nbase + k * GCH, GCH)], xrow)

        def ab(j, carry):
            for t in range(HID // 16):
                grow[j, pl.ds(t * 16, 16)] = (
                    grow[j, pl.ds(t * 16, 16)] + xrow[j, pl.ds(t * 16, 16)])
            return carry
        lax.fori_loop(0, GCH, ab, 0)
        pltpu.sync_copy(grow, h1pre_hbm.at[pl.ds(nbase + k * GCH, GCH)])


_sc_deg = pl.kernel(
    _deg_sc_body,
    out_type=(jax.ShapeDtypeStruct((2 * N_PAD, HID), jnp.float32),
              jax.ShapeDtypeStruct((N_PAD, HID), jnp.float32)),
    mesh=_sc_mesh,
    scratch_types=[
        pltpu.VMEM((CH,), jnp.int32),
        pltpu.VMEM((CH,), jnp.int32),
        pltpu.VMEM((CH, HID), jnp.float32),
        pltpu.VMEM((GCH,), jnp.int32),
        pltpu.VMEM((GCH, HID), jnp.float32),
        pltpu.VMEM((GCH, HID), jnp.float32),
        pltpu.VMEM_SHARED((N_PAD, HID), jnp.float32),
        pltpu.SemaphoreType.DMA,
    ],
)


def _deg_body(d0_ref, d1_ref, h_ref, dinv_ref, hs_ref):
    dinv = lax.rsqrt(d0_ref[:, :1] + d1_ref[:, :1] + 1.0)
    dinv_ref[...] = dinv
    hs_ref[...] = h_ref[...] * dinv


def _deg_reduce(degp, h1pre, br=1024):
    nblk = N_PAD // br
    return pl.pallas_call(
        _deg_body,
        grid=(nblk,),
        in_specs=[pl.BlockSpec((br, HID), lambda i: (i, 0)),
                  pl.BlockSpec((br, HID), lambda i: (i + nblk, 0)),
                  pl.BlockSpec((br, HID), lambda i: (i, 0))],
        out_specs=(pl.BlockSpec((br, 1), lambda i: (i, 0)),
                   pl.BlockSpec((br, HID), lambda i: (i, 0))),
        out_shape=(jax.ShapeDtypeStruct((N_PAD, 1), jnp.float32),
                   jax.ShapeDtypeStruct((N_PAD, HID), jnp.float32)),
    )(degp, degp, h1pre)


def _agg_body(hs_hbm, srcm_hbm, dstm_hbm, accp_hbm,
              sidx, didx, sidx_b, didx_b, rows_a, rows_b, acc, sem_a, sem_b):
    c = lax.axis_index("c")
    s = lax.axis_index("s")

    def zb(j, carry):
        for t in range(HID // 16):
            rows_a[j, pl.ds(t * 16, 16)] = jnp.zeros((16,), jnp.float32)
        return carry
    lax.fori_loop(0, CH, zb, 0)

    def zc(r, carry):
        pltpu.sync_copy(rows_a, acc.at[pl.ds(s * RPS + r * CH, CH)])
        return carry
    lax.fori_loop(0, RPS // CH, zc, 0)
    plsc.subcore_barrier()

    ebase = c * (E_PAD // 2) + s * EPW

    def eb(i, carry):
        base = ebase + 4 * i * CH
        pltpu.sync_copy(srcm_hbm.at[pl.ds(base, CH)], sidx)
        cp0 = pltpu.async_copy(hs_hbm.at[sidx], rows_a, sem_a)
        pltpu.sync_copy(srcm_hbm.at[pl.ds(base + CH, CH)], sidx_b)
        cp1 = pltpu.async_copy(hs_hbm.at[sidx_b], rows_b, sem_b)
        pltpu.sync_copy(dstm_hbm.at[pl.ds(base, CH)], didx)
        cp0.wait()
        pltpu.sync_copy(rows_a, acc.at[didx], add=True)
        pltpu.sync_copy(srcm_hbm.at[pl.ds(base + 2 * CH, CH)], sidx)
        cp2 = pltpu.async_copy(hs_hbm.at[sidx], rows_a, sem_a)
        pltpu.sync_copy(dstm_hbm.at[pl.ds(base + CH, CH)], didx_b)
        cp1.wait()
        pltpu.sync_copy(rows_b, acc.at[didx_b], add=True)
        pltpu.sync_copy(srcm_hbm.at[pl.ds(base + 3 * CH, CH)], sidx_b)
        cp3 = pltpu.async_copy(hs_hbm.at[sidx_b], rows_b, sem_b)
        pltpu.sync_copy(dstm_hbm.at[pl.ds(base + 2 * CH, CH)], didx)
        cp2.wait()
        pltpu.sync_copy(rows_a, acc.at[didx], add=True)
        pltpu.sync_copy(dstm_hbm.at[pl.ds(base + 3 * CH, CH)], didx_b)
        cp3.wait()
        pltpu.sync_copy(rows_b, acc.at[didx_b], add=True)
        return carry
    lax.fori_loop(0, CPW // 4, eb, 0)

    plsc.subcore_barrier()
    pltpu.sync_copy(acc.at[pl.ds(s * RPS, RPS)],
                    accp_hbm.at[pl.ds(c * N_PAD + s * RPS, RPS)])


_sc_agg = pl.kernel(
    _agg_body,
    out_type=jax.ShapeDtypeStruct((2 * N_PAD, HID), jnp.float32),
    mesh=_sc_mesh,
    scratch_types=[
        pltpu.VMEM((CH,), jnp.int32),
        pltpu.VMEM((CH,), jnp.int32),
        pltpu.VMEM((CH,), jnp.int32),
        pltpu.VMEM((CH,), jnp.int32),
        pltpu.VMEM((CH, HID), jnp.float32),
        pltpu.VMEM((CH, HID), jnp.float32),
        pltpu.VMEM_SHARED((N_PAD, HID), jnp.float32),
        pltpu.SemaphoreType.DMA,
        pltpu.SemaphoreType.DMA,
    ],
)


def _conv_body(a0_ref, a1_ref, hs_ref, d_ref, b_ref, w_ref, o_ref):
    dinv = d_ref[...]
    z = jnp.maximum(
        dinv * (a0_ref[...] + a1_ref[...] + hs_ref[...]) + b_ref[...], 0.0)
    o_ref[...] = jnp.dot(z, w_ref[...],
                         preferred_element_type=jnp.float32) * dinv


def _conv_mm(accp, hs, dinv_col, brow, w, br=1024):
    nblk = N_PAD // br
    return pl.pallas_call(
        _conv_body,
        grid=(nblk,),
        in_specs=[pl.BlockSpec((br, HID), lambda i: (i, 0)),
                  pl.BlockSpec((br, HID), lambda i: (i + nblk, 0)),
                  pl.BlockSpec((br, HID), lambda i: (i, 0)),
                  pl.BlockSpec((br, 1), lambda i: (i, 0)),
                  pl.BlockSpec((1, HID), lambda i: (0, 0)),
                  pl.BlockSpec((HID, HID), lambda i: (0, 0))],
        out_specs=pl.BlockSpec((br, HID), lambda i: (i, 0)),
        out_shape=jax.ShapeDtypeStruct((N_PAD, HID), jnp.float32),
    )(accp, accp, hs, dinv_col, brow, w)


NBLK2 = N_PAD // 128


def _final_body(acc_ref, hs_ref, dinv_ref, batch_ref, b2d_ref, uv_ref, b2_ref,
                wm1_ref, bm1_ref, wm2_ref, bm2_ref, o_ref, z2_ref, bm_ref):
    dinv = dinv_ref[...]
    z2_ref[...] = jnp.maximum(
        dinv * (acc_ref[:N_PAD, :] + acc_ref[N_PAD:, :] + hs_ref[...])
        + b2_ref[...], 0.0)

    for b in range(NBLK2):
        bm_ref[b:b + 1, :] = jnp.max(
            z2_ref[b * 128:(b + 1) * 128, :], axis=0, keepdims=True)

    gsel = lax.broadcasted_iota(jnp.int32, (G, 1), 0)
    sub8 = lax.broadcasted_iota(jnp.int32, (8, 1), 0)
    blockid = lax.broadcasted_iota(jnp.int32, (NBLK2, 1), 0)

    def _row_at(i):
        base = pl.multiple_of((i // 8) * 8, 8)
        blk = z2_ref[pl.ds(base, 8), :]
        return jnp.max(jnp.where(sub8 == i - base, blk, -jnp.inf), axis=0)

    def _masked_block_max(row0, g):
        blk = z2_ref[pl.ds(row0, 128), :]
        m = batch_ref[pl.ds(row0, 128), :] == g
        return jnp.max(jnp.where(m, blk, -jnp.inf), axis=0)

    def gloop(g, feats):
        batch2d = b2d_ref[...]
        ptr_g = jnp.sum((batch2d < g).astype(jnp.int32))
        ptr_n = jnp.sum((batch2d < g + 1).astype(jnp.int32))
        fullmask = (blockid * 128 >= ptr_g) & ((blockid + 1) * 128 <= ptr_n)
        pool = jnp.max(jnp.where(fullmask, bm_ref[...], -jnp.inf), axis=0)
        lo = pl.multiple_of((ptr_g // 128) * 128, 128)
        hi = pl.multiple_of((jnp.maximum(ptr_n, 1) - 1) // 128 * 128, 128)
        pool = jnp.maximum(pool, _masked_block_max(lo, g))
        pool = jnp.maximum(pool, _masked_block_max(hi, g))
        iu = jnp.minimum(ptr_g + uv_ref[2 * g], N - 1)
        iv = jnp.minimum(ptr_g + uv_ref[2 * g + 1], N - 1)
        hu = _row_at(iu)
        hv = _row_at(iv)
        row = jnp.concatenate(
            [hu, hv, jnp.abs(hu - hv), hu * hv, pool])[None, :]
        return jnp.where(gsel == g, row, feats)

    feats = lax.fori_loop(0, G, gloop,
                          jnp.zeros((G, 5 * HID), jnp.float32))

    hidden = jnp.maximum(
        jnp.dot(feats, wm1_ref[...],
                preferred_element_type=jnp.float32) + bm1_ref[...], 0.0)
    o_ref[...] = jnp.dot(hidden, wm2_ref[...],
                         preferred_element_type=jnp.float32) + bm2_ref[...]


def _final(accp, hs, dinv_col, batch_col, batch2d, uv, b2r, wm1, bm1r, wm2,
           bm2r):
    return pl.pallas_call(
        _final_body,
        in_specs=[
            pl.BlockSpec((2 * N_PAD, HID), lambda: (0, 0)),
            pl.BlockSpec((N_PAD, HID), lambda: (0, 0)),
            pl.BlockSpec((N_PAD, 1), lambda: (0, 0)),
            pl.BlockSpec((N_PAD, 1), lambda: (0, 0)),
            pl.BlockSpec((NBLK2, 128), lambda: (0, 0)),
            pl.BlockSpec(memory_space=pltpu.SMEM),
            pl.BlockSpec((1, HID), lambda: (0, 0)),
            pl.BlockSpec((5 * HID, HID), lambda: (0, 0)),
            pl.BlockSpec((1, HID), lambda: (0, 0)),
            pl.BlockSpec((HID, 2), lambda: (0, 0)),
            pl.BlockSpec((1, 2), lambda: (0, 0)),
        ],
        out_specs=pl.BlockSpec((G, 2), lambda: (0, 0)),
        out_shape=jax.ShapeDtypeStruct((G, 2), jnp.float32),
        scratch_shapes=[pltpu.VMEM((N_PAD, HID), jnp.float32),
                        pltpu.VMEM((NBLK2, 128), jnp.float32)],
    )(accp, hs, dinv_col, batch_col, batch2d, uv, b2r, wm1, bm1r, wm2, bm2r)


def kernel(x, edge_index, drnl, batch, target_local, emb, W1, b1, W2, b2,
           Wm1, bm1, Wm2, bm2):
    src = edge_index[0].astype(jnp.int32)
    dst = edge_index[1].astype(jnp.int32)
    pad_fill = N + (jnp.arange(E_PAD - E, dtype=jnp.int32) % (N_PAD - N))
    src_p = jnp.concatenate([src, pad_fill])
    dst_p = jnp.concatenate([dst, pad_fill])
    drnl_p = jnp.zeros((N_PAD,), jnp.int32).at[:N].set(drnl.astype(jnp.int32))
    batch_col = jnp.full((N_PAD, 1), G, jnp.int32).at[:N, 0].set(
        batch.astype(jnp.int32))
    batch2d = batch_col.reshape(NBLK2, 128)
    x_p = jnp.zeros((N_PAD, D), jnp.float32).at[:N].set(x)
    uv = target_local.astype(jnp.int32)

    embw = _matmul(emb, W1[D:], emb.shape[0])
    xw = _matmul(x_p, W1[:D], 1024)
    degp, h1pre = _sc_deg(dst_p, drnl_p, embw, xw)
    dinv_col, hs1 = _deg_reduce(degp, h1pre)
    accp1 = _sc_agg(hs1, src_p, dst_p)
    hs2 = _conv_mm(accp1, hs1, dinv_col, b1.reshape(1, HID), W2)
    accp2 = _sc_agg(hs2, src_p, dst_p)
    return _final(accp2, hs2, dinv_col, batch_col, batch2d, uv,
                  b2.reshape(1, HID), Wm1, bm1.reshape(1, HID), Wm2,
                  bm2.reshape(1, 2))


# speedup vs baseline: 3.1700x; 1.1422x over previous
import functools

import jax
import jax.numpy as jnp
from jax import lax
from jax.experimental import pallas as pl
from jax.experimental.pallas import tpu as pltpu
from jax.experimental.pallas import tpu_sc as plsc

N = 10000
E = 320000
D = 128
HID = 128
G = 64

NC, NS = 2, 16
NW = NC * NS
N_PAD = 10240
PAD_NODE = N_PAD - 1
CH = 128
CPW = 80
SCH = 8
EPW = CPW * CH
E_PAD = NW * EPW
EROWS = E_PAD // CH
RPW = N_PAD // NW
RPS = N_PAD // NS
GCH = 80

_sc_mesh = plsc.VectorSubcoreMesh(
    core_axis_name="c", subcore_axis_name="s", num_cores=NC, num_subcores=NS)


def _mm_body(a_ref, w_ref, o_ref):
    o_ref[...] = jnp.dot(a_ref[...], w_ref[...],
                         preferred_element_type=jnp.float32)


def _matmul(a, w, br):
    m, k = a.shape
    n = w.shape[1]
    return pl.pallas_call(
        _mm_body,
        grid=(m // br,),
        in_specs=[pl.BlockSpec((br, k), lambda i: (i, 0)),
                  pl.BlockSpec((k, n), lambda i: (0, 0))],
        out_specs=pl.BlockSpec((br, n), lambda i: (i, 0)),
        out_shape=jax.ShapeDtypeStruct((m, n), jnp.float32),
    )(a, w)


def _deg_sc_body(dstm_hbm, drnl_hbm, embw_hbm, xw_hbm, degp_hbm, h1pre_hbm,
                 didx, didx_b, dstage, ones_v, gidx, grow, xrow, acc, gsem):
    c = lax.axis_index("c")
    s = lax.axis_index("s")

    def zb(j, carry):
        for t in range(HID // 16):
            ones_v[j, pl.ds(t * 16, 16)] = jnp.zeros((16,), jnp.float32)
        return carry
    lax.fori_loop(0, CH, zb, 0)

    def zc(r, carry):
        pltpu.sync_copy(ones_v, acc.at[pl.ds(s * RPS + r * CH, CH)])
        return carry
    lax.fori_loop(0, RPS // CH, zc, 0)

    def ob(j, carry):
        for t in range(HID // 16):
            ones_v[j, pl.ds(t * 16, 16)] = jnp.ones((16,), jnp.float32)
        return carry
    lax.fori_loop(0, CH, ob, 0)
    plsc.subcore_barrier()

    ebase = c * (E_PAD // 2) + s * EPW
    pltpu.sync_copy(dstm_hbm.at[pl.ds(ebase, EPW)], dstage)

    def eb(i, carry):
        def cpidx(dref, off):
            for t in range(CH // 16):
                dref[pl.ds(t * 16, 16)] = dstage[pl.ds(off + t * 16, 16)]
        cpidx(didx, 2 * i * CH)
        pltpu.sync_copy(ones_v, acc.at[didx], add=True)
        cpidx(didx_b, (2 * i + 1) * CH)
        pltpu.sync_copy(ones_v, acc.at[didx_b], add=True)
        return carry
    lax.fori_loop(0, CPW // 2, eb, 0)

    plsc.subcore_barrier()
    pltpu.sync_copy(acc.at[pl.ds(s * RPS, RPS)],
                    degp_hbm.at[pl.ds(c * N_PAD + s * RPS, RPS)])

    wid = s * NC + c
    nbase = wid * RPW
    for k in range(RPW // GCH):
        pltpu.sync_copy(drnl_hbm.at[pl.ds(nbase + k * GCH, GCH)], gidx)
        pltpu.async_copy(embw_hbm.at[gidx], grow, gsem).wait()
        pltpu.sync_copy(xw_hbm.at[pl.ds(nbase + k * GCH, GCH)], xrow)

        def ab(j, carry):
            for t in range(HID // 16):
                grow[j, pl.ds(t * 16, 16)] = (
                    grow[j, pl.ds(t * 16, 16)] + xrow[j, pl.ds(t * 16, 16)])
            return carry
        lax.fori_loop(0, GCH, ab, 0)
        pltpu.sync_copy(grow, h1pre_hbm.at[pl.ds(nbase + k * GCH, GCH)])


_sc_deg = pl.kernel(
    _deg_sc_body,
    out_type=(jax.ShapeDtypeStruct((2 * N_PAD, HID), jnp.float32),
              jax.ShapeDtypeStruct((N_PAD, HID), jnp.float32)),
    mesh=_sc_mesh,
    scratch_types=[
        pltpu.VMEM((CH,), jnp.int32),
        pltpu.VMEM((CH,), jnp.int32),
        pltpu.VMEM((EPW,), jnp.int32),
        pltpu.VMEM((CH, HID), jnp.float32),
        pltpu.VMEM((GCH,), jnp.int32),
        pltpu.VMEM((GCH, HID), jnp.float32),
        pltpu.VMEM((GCH, HID), jnp.float32),
        pltpu.VMEM_SHARED((N_PAD, HID), jnp.float32),
        pltpu.SemaphoreType.DMA,
    ],
)


def _deg_body(d0_ref, d1_ref, h_ref, dinv_ref, hs_ref):
    dinv = lax.rsqrt(d0_ref[:, :1] + d1_ref[:, :1] + 1.0)
    dinv_ref[...] = dinv
    hs_ref[...] = h_ref[...] * dinv


def _deg_reduce(degp, h1pre, br=1024):
    nblk = N_PAD // br
    return pl.pallas_call(
        _deg_body,
        grid=(nblk,),
        in_specs=[pl.BlockSpec((br, HID), lambda i: (i, 0)),
                  pl.BlockSpec((br, HID), lambda i: (i + nblk, 0)),
                  pl.BlockSpec((br, HID), lambda i: (i, 0))],
        out_specs=(pl.BlockSpec((br, 1), lambda i: (i, 0)),
                   pl.BlockSpec((br, HID), lambda i: (i, 0))),
        out_shape=(jax.ShapeDtypeStruct((N_PAD, 1), jnp.float32),
                   jax.ShapeDtypeStruct((N_PAD, HID), jnp.float32)),
    )(degp, degp, h1pre)


def _agg_body(hs_hbm, srcm_hbm, dstm_hbm, accp_hbm,
              sidx, didx, sidx_b, didx_b, sstage, dstage, rows_a, rows_b,
              acc, sem_a, sem_b):
    c = lax.axis_index("c")
    s = lax.axis_index("s")

    def zb(j, carry):
        for t in range(HID // 16):
            rows_a[j, pl.ds(t * 16, 16)] = jnp.zeros((16,), jnp.float32)
        return carry
    lax.fori_loop(0, CH, zb, 0)

    def zc(r, carry):
        pltpu.sync_copy(rows_a, acc.at[pl.ds(s * RPS + r * CH, CH)])
        return carry
    lax.fori_loop(0, RPS // CH, zc, 0)
    plsc.subcore_barrier()

    ebase = c * (E_PAD // 2) + s * EPW
    HEPW = EPW // 2

    def hloop(h, hcarry):
        pltpu.sync_copy(srcm_hbm.at[pl.ds(ebase + h * HEPW, HEPW)], sstage)
        pltpu.sync_copy(dstm_hbm.at[pl.ds(ebase + h * HEPW, HEPW)], dstage)

        def cps(dref, off):
            for t in range(CH // 16):
                dref[pl.ds(t * 16, 16)] = sstage[pl.ds(off + t * 16, 16)]

        def cpd(dref, off):
            for t in range(CH // 16):
                dref[pl.ds(t * 16, 16)] = dstage[pl.ds(off + t * 16, 16)]

        def eb(i, carry):
            base = 4 * i * CH
            cps(sidx, base)
            cp0 = pltpu.async_copy(hs_hbm.at[sidx], rows_a, sem_a)
            cps(sidx_b, base + CH)
            cp1 = pltpu.async_copy(hs_hbm.at[sidx_b], rows_b, sem_b)
            cpd(didx, base)
            cp0.wait()
            pltpu.sync_copy(rows_a, acc.at[didx], add=True)
            cps(sidx, base + 2 * CH)
            cp2 = pltpu.async_copy(hs_hbm.at[sidx], rows_a, sem_a)
            cpd(didx_b, base + CH)
            cp1.wait()
            pltpu.sync_copy(rows_b, acc.at[didx_b], add=True)
            cps(sidx_b, base + 3 * CH)
            cp3 = pltpu.async_copy(hs_hbm.at[sidx_b], rows_b, sem_b)
            cpd(didx, base + 2 * CH)
            cp2.wait()
            pltpu.sync_copy(rows_a, acc.at[didx], add=True)
            cpd(didx_b, base + 3 * CH)
            cp3.wait()
            pltpu.sync_copy(rows_b, acc.at[didx_b], add=True)
            return carry
        lax.fori_loop(0, HEPW // (4 * CH), eb, 0)
        return hcarry
    lax.fori_loop(0, 2, hloop, 0)

    plsc.subcore_barrier()
    pltpu.sync_copy(acc.at[pl.ds(s * RPS, RPS)],
                    accp_hbm.at[pl.ds(c * N_PAD + s * RPS, RPS)])


_sc_agg = pl.kernel(
    _agg_body,
    out_type=jax.ShapeDtypeStruct((2 * N_PAD, HID), jnp.float32),
    mesh=_sc_mesh,
    scratch_types=[
        pltpu.VMEM((CH,), jnp.int32),
        pltpu.VMEM((CH,), jnp.int32),
        pltpu.VMEM((CH,), jnp.int32),
        pltpu.VMEM((CH,), jnp.int32),
        pltpu.VMEM((EPW // 2,), jnp.int32),
        pltpu.VMEM((EPW // 2,), jnp.int32),
        pltpu.VMEM((CH, HID), jnp.float32),
        pltpu.VMEM((CH, HID), jnp.float32),
        pltpu.VMEM_SHARED((N_PAD, HID), jnp.float32),
        pltpu.SemaphoreType.DMA,
        pltpu.SemaphoreType.DMA,
    ],
)


def _conv_body(a0_ref, a1_ref, hs_ref, d_ref, b_ref, w_ref, o_ref):
    dinv = d_ref[...]
    z = jnp.maximum(
        dinv * (a0_ref[...] + a1_ref[...] + hs_ref[...]) + b_ref[...], 0.0)
    o_ref[...] = jnp.dot(z, w_ref[...],
                         preferred_element_type=jnp.float32) * dinv


def _conv_mm(accp, hs, dinv_col, brow, w, br=1024):
    nblk = N_PAD // br
    return pl.pallas_call(
        _conv_body,
        grid=(nblk,),
        in_specs=[pl.BlockSpec((br, HID), lambda i: (i, 0)),
                  pl.BlockSpec((br, HID), lambda i: (i + nblk, 0)),
                  pl.BlockSpec((br, HID), lambda i: (i, 0)),
                  pl.BlockSpec((br, 1), lambda i: (i, 0)),
                  pl.BlockSpec((1, HID), lambda i: (0, 0)),
                  pl.BlockSpec((HID, HID), lambda i: (0, 0))],
        out_specs=pl.BlockSpec((br, HID), lambda i: (i, 0)),
        out_shape=jax.ShapeDtypeStruct((N_PAD, HID), jnp.float32),
    )(accp, accp, hs, dinv_col, brow, w)


NBLK2 = N_PAD // 128


def _final_body(acc_ref, hs_ref, dinv_ref, batch_ref, b2d_ref, uv_ref, b2_ref,
                wm1_ref, bm1_ref, wm2_ref, bm2_ref, o_ref, z2_ref, bm_ref):
    dinv = dinv_ref[...]
    z2_ref[...] = jnp.maximum(
        dinv * (acc_ref[:N_PAD, :] + acc_ref[N_PAD:, :] + hs_ref[...])
        + b2_ref[...], 0.0)

    for b in range(NBLK2):
        bm_ref[b:b + 1, :] = jnp.max(
            z2_ref[b * 128:(b + 1) * 128, :], axis=0, keepdims=True)

    gsel = lax.broadcasted_iota(jnp.int32, (G, 1), 0)
    sub8 = lax.broadcasted_iota(jnp.int32, (8, 1), 0)
    blockid = lax.broadcasted_iota(jnp.int32, (NBLK2, 1), 0)

    def _row_at(i):
        base = pl.multiple_of((i // 8) * 8, 8)
        blk = z2_ref[pl.ds(base, 8), :]
        return jnp.max(jnp.where(sub8 == i - base, blk, -jnp.inf), axis=0)

    def _masked_block_max(row0, g):
        blk = z2_ref[pl.ds(row0, 128), :]
        m = batch_ref[pl.ds(row0, 128), :] == g
        return jnp.max(jnp.where(m, blk, -jnp.inf), axis=0)

    def gloop(g, feats):
        batch2d = b2d_ref[...]
        ptr_g = jnp.sum((batch2d < g).astype(jnp.int32))
        ptr_n = jnp.sum((batch2d < g + 1).astype(jnp.int32))
        fullmask = (blockid * 128 >= ptr_g) & ((blockid + 1) * 128 <= ptr_n)
        pool = jnp.max(jnp.where(fullmask, bm_ref[...], -jnp.inf), axis=0)
        lo = pl.multiple_of((ptr_g // 128) * 128, 128)
        hi = pl.multiple_of((jnp.maximum(ptr_n, 1) - 1) // 128 * 128, 128)
        pool = jnp.maximum(pool, _masked_block_max(lo, g))
        pool = jnp.maximum(pool, _masked_block_max(hi, g))
        iu = jnp.minimum(ptr_g + uv_ref[2 * g], N - 1)
        iv = jnp.minimum(ptr_g + uv_ref[2 * g + 1], N - 1)
        hu = _row_at(iu)
        hv = _row_at(iv)
        row = jnp.concatenate(
            [hu, hv, jnp.abs(hu - hv), hu * hv, pool])[None, :]
        return jnp.where(gsel == g, row, feats)

    feats = lax.fori_loop(0, G, gloop,
                          jnp.zeros((G, 5 * HID), jnp.float32))

    hidden = jnp.maximum(
        jnp.dot(feats, wm1_ref[...],
                preferred_element_type=jnp.float32) + bm1_ref[...], 0.0)
    o_ref[...] = jnp.dot(hidden, wm2_ref[...],
                         preferred_element_type=jnp.float32) + bm2_ref[...]


def _final(accp, hs, dinv_col, batch_col, batch2d, uv, b2r, wm1, bm1r, wm2,
           bm2r):
    return pl.pallas_call(
        _final_body,
        in_specs=[
            pl.BlockSpec((2 * N_PAD, HID), lambda: (0, 0)),
            pl.BlockSpec((N_PAD, HID), lambda: (0, 0)),
            pl.BlockSpec((N_PAD, 1), lambda: (0, 0)),
            pl.BlockSpec((N_PAD, 1), lambda: (0, 0)),
            pl.BlockSpec((NBLK2, 128), lambda: (0, 0)),
            pl.BlockSpec(memory_space=pltpu.SMEM),
            pl.BlockSpec((1, HID), lambda: (0, 0)),
            pl.BlockSpec((5 * HID, HID), lambda: (0, 0)),
            pl.BlockSpec((1, HID), lambda: (0, 0)),
            pl.BlockSpec((HID, 2), lambda: (0, 0)),
            pl.BlockSpec((1, 2), lambda: (0, 0)),
        ],
        out_specs=pl.BlockSpec((G, 2), lambda: (0, 0)),
        out_shape=jax.ShapeDtypeStruct((G, 2), jnp.float32),
        scratch_shapes=[pltpu.VMEM((N_PAD, HID), jnp.float32),
                        pltpu.VMEM((NBLK2, 128), jnp.float32)],
    )(accp, hs, dinv_col, batch_col, batch2d, uv, b2r, wm1, bm1r, wm2, bm2r)


def kernel(x, edge_index, drnl, batch, target_local, emb, W1, b1, W2, b2,
           Wm1, bm1, Wm2, bm2):
    src = edge_index[0].astype(jnp.int32)
    dst = edge_index[1].astype(jnp.int32)
    pad_fill = N + (jnp.arange(E_PAD - E, dtype=jnp.int32) % (N_PAD - N))
    src_p = jnp.concatenate([src, pad_fill])
    dst_p = jnp.concatenate([dst, pad_fill])
    drnl_p = jnp.zeros((N_PAD,), jnp.int32).at[:N].set(drnl.astype(jnp.int32))
    batch_col = jnp.full((N_PAD, 1), G, jnp.int32).at[:N, 0].set(
        batch.astype(jnp.int32))
    batch2d = batch_col.reshape(NBLK2, 128)
    x_p = jnp.zeros((N_PAD, D), jnp.float32).at[:N].set(x)
    uv = target_local.astype(jnp.int32)

    embw = _matmul(emb, W1[D:], emb.shape[0])
    xw = _matmul(x_p, W1[:D], 1024)
    degp, h1pre = _sc_deg(dst_p, drnl_p, embw, xw)
    dinv_col, hs1 = _deg_reduce(degp, h1pre)
    accp1 = _sc_agg(hs1, src_p, dst_p)
    hs2 = _conv_mm(accp1, hs1, dinv_col, b1.reshape(1, HID), W2)
    accp2 = _sc_agg(hs2, src_p, dst_p)
    return _final(accp2, hs2, dinv_col, batch_col, batch2d, uv,
                  b2.reshape(1, HID), Wm1, bm1.reshape(1, HID), Wm2,
                  bm2.reshape(1, 2))
